# baseline (speedup 1.0000x reference)
import functools

import jax
import jax.numpy as jnp
from jax import lax
from jax.experimental import pallas as pl
from jax.experimental.pallas import tpu as pltpu
from jax.experimental.pallas import tpu_sc as plsc


_INFO = plsc.get_sparse_core_info()
_NC = _INFO.num_cores
_NS = _INFO.num_subcores
_NW = _NC * _NS


@functools.lru_cache(maxsize=None)
def _make_gather(N: int, D: int, CK: int):
    b_per_w = N // _NW
    nchunk = b_per_w // CK
    nround = nchunk // 2
    npair = nround // 2
    mesh = plsc.VectorSubcoreMesh(core_axis_name="c", subcore_axis_name="s")

    @functools.partial(
        pl.kernel,
        mesh=mesh,
        out_type=jax.ShapeDtypeStruct((N, D), jnp.float32),
        scratch_types=[
            pltpu.VMEM((nchunk, CK), jnp.int32),
            pltpu.VMEM((CK, D), jnp.float32),
            pltpu.VMEM((CK, D), jnp.float32),
            pltpu.VMEM((CK, D), jnp.float32),
            pltpu.VMEM((CK, D), jnp.float32),
            pltpu.SemaphoreType.DMA,
            pltpu.SemaphoreType.DMA,
            pltpu.SemaphoreType.DMA,
            pltpu.SemaphoreType.DMA,
            pltpu.SemaphoreType.DMA,
            pltpu.SemaphoreType.DMA,
            pltpu.SemaphoreType.DMA,
            pltpu.SemaphoreType.DMA,
        ],
    )
    def gather_kernel(
        table_hbm, idx_hbm, out_hbm, idx_v,
        a0, a1, b0, b1, ga0, ga1, gb0, gb1, sa0, sa1, sb0, sb1,
    ):
        wid = lax.axis_index("s") * _NC + lax.axis_index("c")
        base = wid * b_per_w
        pltpu.sync_copy(idx_hbm.at[wid], idx_v)

        def orow(g):
            return out_hbm.at[pl.ds(base + g * CK, CK)]

        def gath(g, buf, sem):
            return pltpu.async_copy(table_hbm.at[idx_v.at[g]], buf, sem)

        def gath_wait(g, buf, sem):
            pltpu.make_async_copy(table_hbm.at[idx_v.at[g]], buf, sem).wait()

        def scat(g, buf, sem):
            return pltpu.async_copy(buf, orow(g), sem)

        def scat_wait(g, buf, sem):
            pltpu.make_async_copy(buf, orow(g), sem).wait()

        gath(0, a0, ga0)
        gath(1, a1, ga1)
        scat(2, b0, sb0)
        scat(3, b1, sb1)

        def body(j, _):
            ra = 2 * j
            rb = ra + 1
            ca, cb, cn = 2 * ra, 2 * rb, 2 * (rb + 1)
            gath_wait(ca, a0, ga0)
            gath_wait(ca + 1, a1, ga1)
            scat_wait(cb, b0, sb0)
            scat_wait(cb + 1, b1, sb1)
            gath(cb, b0, gb0)
            gath(cb + 1, b1, gb1)
            scat(ca, a0, sa0)
            scat(ca + 1, a1, sa1)
            gath_wait(cb, b0, gb0)
            gath_wait(cb + 1, b1, gb1)
            scat_wait(ca, a0, sa0)
            scat_wait(ca + 1, a1, sa1)
            gath(cn % nchunk, a0, ga0)
            gath((cn + 1) % nchunk, a1, ga1)
            scat(cb, b0, sb0)
            scat(cb + 1, b1, sb1)
            return 0

        lax.fori_loop(0, npair, body, 0)
        gath_wait(0, a0, ga0)
        gath_wait(1, a1, ga1)
        scat_wait(nchunk - 2, b0, sb0)
        scat_wait(nchunk - 1, b1, sb1)

    return gather_kernel


def kernel(token_indices, table):
    B, T = token_indices.shape
    V, D = table.shape
    N = B * T
    CK = 2
    idx = token_indices.astype(jnp.int32).reshape(_NW, (N // _NW) // CK, CK)
    out = _make_gather(N, D, CK)(table, idx)
    return out.reshape(B, T, D)
